# Initial kernel scaffold; baseline (speedup 1.0000x reference)
import jax
import jax.numpy as jnp
from jax.experimental import pallas as pl


def kernel(x, edge_index, batch, W_l, b_l, W_r, b_r, att):
    raise NotImplementedError("write your pallas kernel here")


# speedup vs baseline: 1.4483x; 1.4483x over previous
import jax
import jax.numpy as jnp
from jax import lax
from jax.experimental import pallas as pl
from jax.experimental.pallas import tpu as pltpu

B, N, C_IN, D, K = 64, 128, 64, 32, 20
_NEG = float("-inf")


def _body(x_ref, xT_ref, wr_ref, wlT_ref, br_ref, blT_ref, attv_ref, atts_ref,
          vals_ref, idx_ref):
    b = pl.program_id(0)
    xb = x_ref[0]
    xTb = xT_ref[0]

    xr = jnp.dot(xb, wr_ref[...], preferred_element_type=jnp.float32) + br_ref[...]
    xlT = jnp.dot(wlT_ref[...], xTb, preferred_element_type=jnp.float32) + blT_ref[...]

    attv = attv_ref[...]
    sr = jnp.sum(xr * attv, axis=1, keepdims=True)
    slT = jnp.sum(xlT * attv.reshape(D, 1), axis=0, keepdims=True)
    alpha = 0.2 * (sr + slT)
    for d in range(D):
        c = 0.8 * atts_ref[0, d]
        alpha = alpha + c * jnp.maximum(xr[:, d:d + 1] + xlT[d:d + 1, :], 0.0)

    alpha = jnp.where(jnp.isfinite(alpha), alpha, 0.0)

    jiota = lax.broadcasted_iota(jnp.int32, (N, N), 0)
    vals_rows = []
    idx_rows = []
    for _ in range(K):
        m = jnp.max(alpha, axis=0, keepdims=True)
        jsel = jnp.min(jnp.where(alpha == m, jiota, N), axis=0, keepdims=True)
        vals_rows.append(m)
        idx_rows.append(jsel)
        alpha = jnp.where(jiota == jsel, _NEG, alpha)

    vals = jnp.concatenate(vals_rows, axis=0)
    idx = jnp.concatenate(idx_rows, axis=0)

    e = jnp.exp(vals - vals[0:1, :])
    vals_ref[0] = e / jnp.sum(e, axis=0, keepdims=True)
    idx_ref[0] = idx + b * N


def kernel(x, edge_index, batch, W_l, b_l, W_r, b_r, att):
    del edge_index, batch
    xT = x.transpose(0, 2, 1)
    wlT = W_l.T
    blT = b_l.reshape(D, 1)
    br = b_r.reshape(1, D)
    att_row = att.reshape(1, D)

    vals, idx = pl.pallas_call(
        _body,
        grid=(B,),
        in_specs=[
            pl.BlockSpec((1, N, C_IN), lambda b: (b, 0, 0)),
            pl.BlockSpec((1, C_IN, N), lambda b: (b, 0, 0)),
            pl.BlockSpec((C_IN, D), lambda b: (0, 0)),
            pl.BlockSpec((D, C_IN), lambda b: (0, 0)),
            pl.BlockSpec((1, D), lambda b: (0, 0)),
            pl.BlockSpec((D, 1), lambda b: (0, 0)),
            pl.BlockSpec((1, D), lambda b: (0, 0)),
            pl.BlockSpec(memory_space=pltpu.SMEM),
        ],
        out_specs=[
            pl.BlockSpec((1, K, N), lambda b: (b, 0, 0)),
            pl.BlockSpec((1, K, N), lambda b: (b, 0, 0)),
        ],
        out_shape=[
            jax.ShapeDtypeStruct((B, K, N), jnp.float32),
            jax.ShapeDtypeStruct((B, K, N), jnp.int32),
        ],
    )(x, xT, W_r, wlT, br, blT, att_row, att_row)

    attention = vals.transpose(0, 2, 1).reshape(-1)
    index_j = idx.transpose(0, 2, 1).reshape(1, -1)
    index_i = (jnp.repeat(jnp.arange(N, dtype=jnp.int32), K)[None, :]
               + jnp.arange(B, dtype=jnp.int32)[:, None] * N).reshape(1, -1)
    new_edge_index = jnp.concatenate((index_i, index_j), axis=0)
    return new_edge_index, attention


# speedup vs baseline: 1.9063x; 1.3162x over previous
import jax
import jax.numpy as jnp
from jax import lax
from jax.experimental import pallas as pl
from jax.experimental.pallas import tpu as pltpu

B, N, C_IN, D, K = 64, 128, 64, 32, 20


def _body(xT_ref, wlT_ref, wrT_ref, blT_ref, brT_ref, attv_ref, atts_ref,
          vals_ref, idx_ref):
    b = pl.program_id(0)
    xTb = xT_ref[0]

    xlT = jnp.dot(wlT_ref[...], xTb, preferred_element_type=jnp.float32) + blT_ref[...]
    xrT = jnp.dot(wrT_ref[...], xTb, preferred_element_type=jnp.float32) + brT_ref[...]

    attv = attv_ref[...]
    ones_row = jnp.ones((1, N), dtype=jnp.float32)

    sr_row = 0.2 * jnp.dot(attv, xrT, preferred_element_type=jnp.float32)
    sl_row = 0.2 * jnp.dot(attv, xlT, preferred_element_type=jnp.float32)
    dn = (((0,), (0,)), ((), ()))
    alpha = lax.dot_general(jnp.concatenate([sr_row, ones_row], axis=0),
                            jnp.concatenate([ones_row, sl_row], axis=0),
                            dn, preferred_element_type=jnp.float32)
    for d in range(D):
        c = 0.8 * atts_ref[0, d]
        s_d = lax.dot_general(jnp.concatenate([xrT[d:d + 1, :], ones_row], axis=0),
                              jnp.concatenate([ones_row, xlT[d:d + 1, :]], axis=0),
                              dn, preferred_element_type=jnp.float32)
        alpha = alpha + c * jnp.maximum(s_d, 0.0)

    alpha = jnp.where(jnp.isfinite(alpha), alpha, 0.0)

    _IMIN = jnp.int32(-2**31)
    _M31 = jnp.int32(0x7FFFFFFF)
    jiota = lax.broadcasted_iota(jnp.int32, (N, N), 0)
    u = lax.bitcast_convert_type(alpha, jnp.int32)
    m = u ^ ((u >> 31) & _M31)
    key = (m & jnp.int32(-128)) | (jnp.int32(127) - jiota)

    kmax_rows = []
    for _ in range(K):
        kmax = jnp.max(key, axis=0, keepdims=True)
        kmax_rows.append(kmax)
        key = jnp.where(key == kmax, _IMIN, key)

    kk = jnp.concatenate(kmax_rows, axis=0)
    jsel = jnp.int32(127) - (kk & jnp.int32(127))
    mt = kk & jnp.int32(-128)
    vals = lax.bitcast_convert_type(mt ^ ((mt >> 31) & _M31), jnp.float32)

    e = jnp.exp(vals - vals[0:1, :])
    vals_ref[0] = e / jnp.sum(e, axis=0, keepdims=True)
    idx_ref[0] = jsel + b * N


def kernel(x, edge_index, batch, W_l, b_l, W_r, b_r, att):
    del edge_index, batch
    xT = x.transpose(0, 2, 1)
    wlT = W_l.T
    wrT = W_r.T
    blT = b_l.reshape(D, 1)
    brT = b_r.reshape(D, 1)
    att_row = att.reshape(1, D)

    vals, idx = pl.pallas_call(
        _body,
        grid=(B,),
        in_specs=[
            pl.BlockSpec((1, C_IN, N), lambda b: (b, 0, 0)),
            pl.BlockSpec((D, C_IN), lambda b: (0, 0)),
            pl.BlockSpec((D, C_IN), lambda b: (0, 0)),
            pl.BlockSpec((D, 1), lambda b: (0, 0)),
            pl.BlockSpec((D, 1), lambda b: (0, 0)),
            pl.BlockSpec((1, D), lambda b: (0, 0)),
            pl.BlockSpec(memory_space=pltpu.SMEM),
        ],
        out_specs=[
            pl.BlockSpec((1, K, N), lambda b: (b, 0, 0)),
            pl.BlockSpec((1, K, N), lambda b: (b, 0, 0)),
        ],
        out_shape=[
            jax.ShapeDtypeStruct((B, K, N), jnp.float32),
            jax.ShapeDtypeStruct((B, K, N), jnp.int32),
        ],
    )(xT, wlT, wrT, blT, brT, att_row, att_row)

    attention = vals.transpose(0, 2, 1).reshape(-1)
    index_j = idx.transpose(0, 2, 1).reshape(1, -1)
    index_i = (jnp.repeat(jnp.arange(N, dtype=jnp.int32), K)[None, :]
               + jnp.arange(B, dtype=jnp.int32)[:, None] * N).reshape(1, -1)
    new_edge_index = jnp.concatenate((index_i, index_j), axis=0)
    return new_edge_index, attention


# speedup vs baseline: 2.0562x; 1.0787x over previous
import jax
import jax.numpy as jnp
from jax import lax
from jax.experimental import pallas as pl
from jax.experimental.pallas import tpu as pltpu

B, N, C_IN, D, K = 64, 128, 64, 32, 20


_DG = 4
_NG = D // _DG
_W = _DG * N


def _body(xT_ref, wlT_ref, wrT_ref, blT_ref, brT_ref, attv_ref, kron_ref,
          catt_ref, vals_ref, idx_ref):
    b = pl.program_id(0)
    xTb = xT_ref[0]

    xlT = jnp.dot(wlT_ref[...], xTb, preferred_element_type=jnp.float32) + blT_ref[...]
    xrT = jnp.dot(wrT_ref[...], xTb, preferred_element_type=jnp.float32) + brT_ref[...]

    attv = attv_ref[...]
    ones_row = jnp.ones((1, N), dtype=jnp.float32)

    sr_row = 0.2 * jnp.dot(attv, xrT, preferred_element_type=jnp.float32)
    sl_row = 0.2 * jnp.dot(attv, xlT, preferred_element_type=jnp.float32)
    dn = (((0,), (0,)), ((), ()))
    alpha = lax.dot_general(jnp.concatenate([sr_row, ones_row], axis=0),
                            jnp.concatenate([ones_row, sl_row], axis=0),
                            dn, preferred_element_type=jnp.float32)

    xlT_flat = xlT.reshape(1, D * N)
    kron = kron_ref[...]
    for g in range(_NG):
        p_g = jnp.concatenate([xrT[g * _DG:(g + 1) * _DG, :], ones_row], axis=0)
        q_g = jnp.concatenate([kron, xlT_flat[:, g * _W:(g + 1) * _W]], axis=0)
        s_g = lax.dot_general(p_g, q_g, dn, preferred_element_type=jnp.float32)
        t_g = jnp.maximum(s_g, 0.0) * catt_ref[:, g * _W:(g + 1) * _W]
        for s in range(_DG):
            alpha = alpha + t_g[:, s * N:(s + 1) * N]

    alpha = jnp.where(jnp.isfinite(alpha), alpha, 0.0)

    _IMIN = jnp.int32(-2**31)
    _M31 = jnp.int32(0x7FFFFFFF)
    jiota = lax.broadcasted_iota(jnp.int32, (N, N), 0)
    u = lax.bitcast_convert_type(alpha, jnp.int32)
    m = u ^ ((u >> 31) & _M31)
    key = (m & jnp.int32(-128)) | (jnp.int32(127) - jiota)

    kmax_rows = []
    for _ in range(K):
        kmax = jnp.max(key, axis=0, keepdims=True)
        kmax_rows.append(kmax)
        key = jnp.where(key == kmax, _IMIN, key)

    kk = jnp.concatenate(kmax_rows, axis=0)
    jsel = jnp.int32(127) - (kk & jnp.int32(127))
    mt = kk & jnp.int32(-128)
    vals = lax.bitcast_convert_type(mt ^ ((mt >> 31) & _M31), jnp.float32)

    e = jnp.exp(vals - vals[0:1, :])
    vals_ref[0] = e / jnp.sum(e, axis=0, keepdims=True)
    idx_ref[0] = jsel + b * N


def kernel(x, edge_index, batch, W_l, b_l, W_r, b_r, att):
    del edge_index, batch
    xT = x.transpose(0, 2, 1)
    wlT = W_l.T
    wrT = W_r.T
    blT = b_l.reshape(D, 1)
    brT = b_r.reshape(D, 1)
    att_row = att.reshape(1, D)
    kron = jnp.kron(jnp.eye(_DG, dtype=jnp.float32),
                    jnp.ones((1, N), dtype=jnp.float32))
    catt = jnp.repeat(0.8 * att.reshape(-1), N)[None, :]

    vals, idx = pl.pallas_call(
        _body,
        grid=(B,),
        in_specs=[
            pl.BlockSpec((1, C_IN, N), lambda b: (b, 0, 0)),
            pl.BlockSpec((D, C_IN), lambda b: (0, 0)),
            pl.BlockSpec((D, C_IN), lambda b: (0, 0)),
            pl.BlockSpec((D, 1), lambda b: (0, 0)),
            pl.BlockSpec((D, 1), lambda b: (0, 0)),
            pl.BlockSpec((1, D), lambda b: (0, 0)),
            pl.BlockSpec((_DG, _W), lambda b: (0, 0)),
            pl.BlockSpec((1, D * N), lambda b: (0, 0)),
        ],
        out_specs=[
            pl.BlockSpec((1, K, N), lambda b: (b, 0, 0)),
            pl.BlockSpec((1, K, N), lambda b: (b, 0, 0)),
        ],
        out_shape=[
            jax.ShapeDtypeStruct((B, K, N), jnp.float32),
            jax.ShapeDtypeStruct((B, K, N), jnp.int32),
        ],
    )(xT, wlT, wrT, blT, brT, att_row, kron, catt)

    attention = vals.transpose(0, 2, 1).reshape(-1)
    index_j = idx.transpose(0, 2, 1).reshape(1, -1)
    index_i = (jnp.repeat(jnp.arange(N, dtype=jnp.int32), K)[None, :]
               + jnp.arange(B, dtype=jnp.int32)[:, None] * N).reshape(1, -1)
    new_edge_index = jnp.concatenate((index_i, index_j), axis=0)
    return new_edge_index, attention


# speedup vs baseline: 2.5029x; 1.2172x over previous
import jax
import jax.numpy as jnp
from jax import lax
from jax.experimental import pallas as pl
from jax.experimental.pallas import tpu as pltpu

B, N, C_IN, D, K = 64, 128, 64, 32, 20


_DG = 4
_NG = D // _DG
_W = _DG * N


_SB = 2


def _body(xT_ref, wlT_ref, wrT_ref, blT_ref, brT_ref, attv_ref, kron_ref,
          catt_ref, vals_ref, idx_ref):
    b = pl.program_id(0)
    for s in range(_SB):
        _sample(xT_ref[s], wlT_ref, wrT_ref, blT_ref, brT_ref, attv_ref,
                kron_ref, catt_ref, vals_ref, idx_ref, s, b * _SB + s)


def _sample(xTb, wlT_ref, wrT_ref, blT_ref, brT_ref, attv_ref, kron_ref,
            catt_ref, vals_ref, idx_ref, s, sample_idx):
    xlT = jnp.dot(wlT_ref[...], xTb, preferred_element_type=jnp.float32) + blT_ref[...]
    xrT = jnp.dot(wrT_ref[...], xTb, preferred_element_type=jnp.float32) + brT_ref[...]

    attv = attv_ref[...]
    ones_row = jnp.ones((1, N), dtype=jnp.float32)

    sr_row = 0.2 * jnp.dot(attv, xrT, preferred_element_type=jnp.float32)
    sl_row = 0.2 * jnp.dot(attv, xlT, preferred_element_type=jnp.float32)
    dn = (((0,), (0,)), ((), ()))
    acc0 = lax.dot_general(jnp.concatenate([sr_row, ones_row], axis=0),
                           jnp.concatenate([ones_row, sl_row], axis=0),
                           dn, preferred_element_type=jnp.float32)
    acc1 = jnp.zeros((N, N), dtype=jnp.float32)

    xlT_flat = xlT.reshape(1, D * N)
    kron = kron_ref[...]
    accs = [acc0, acc1]
    for g in range(_NG):
        p_g = jnp.concatenate([xrT[g * _DG:(g + 1) * _DG, :], ones_row], axis=0)
        q_g = jnp.concatenate([kron, xlT_flat[:, g * _W:(g + 1) * _W]], axis=0)
        s_g = lax.dot_general(p_g, q_g, dn, preferred_element_type=jnp.float32)
        t_g = jnp.maximum(s_g, 0.0) * catt_ref[:, g * _W:(g + 1) * _W]
        for t in range(_DG):
            accs[t % 2] = accs[t % 2] + t_g[:, t * N:(t + 1) * N]
    alpha = accs[0] + accs[1]

    alpha = jnp.where(jnp.isfinite(alpha), alpha, 0.0)

    _IMIN = jnp.int32(-2**31)
    _M31 = jnp.int32(0x7FFFFFFF)
    jiota = lax.broadcasted_iota(jnp.int32, (N, N), 0)
    u = lax.bitcast_convert_type(alpha, jnp.int32)
    m = u ^ ((u >> 31) & _M31)
    key = (m & jnp.int32(-128)) | (jnp.int32(127) - jiota)

    kmax_rows = []
    for _ in range(K):
        kmax = jnp.max(key, axis=0, keepdims=True)
        kmax_rows.append(kmax)
        key = jnp.where(key == kmax, _IMIN, key)

    kk = jnp.concatenate(kmax_rows, axis=0)
    jsel = jnp.int32(127) - (kk & jnp.int32(127))
    mt = kk & jnp.int32(-128)
    vals = lax.bitcast_convert_type(mt ^ ((mt >> 31) & _M31), jnp.float32)

    e = jnp.exp(vals - vals[0:1, :])
    vals_ref[s] = e / jnp.sum(e, axis=0, keepdims=True)
    idx_ref[s] = jsel + sample_idx * N


def kernel(x, edge_index, batch, W_l, b_l, W_r, b_r, att):
    del edge_index, batch
    xT = x.transpose(0, 2, 1)
    wlT = W_l.T
    wrT = W_r.T
    blT = b_l.reshape(D, 1)
    brT = b_r.reshape(D, 1)
    att_row = att.reshape(1, D)
    kron = jnp.kron(jnp.eye(_DG, dtype=jnp.float32),
                    jnp.ones((1, N), dtype=jnp.float32))
    catt = jnp.repeat(0.8 * att.reshape(-1), N)[None, :]

    vals, idx = pl.pallas_call(
        _body,
        grid=(B // _SB,),
        in_specs=[
            pl.BlockSpec((_SB, C_IN, N), lambda b: (b, 0, 0)),
            pl.BlockSpec((D, C_IN), lambda b: (0, 0)),
            pl.BlockSpec((D, C_IN), lambda b: (0, 0)),
            pl.BlockSpec((D, 1), lambda b: (0, 0)),
            pl.BlockSpec((D, 1), lambda b: (0, 0)),
            pl.BlockSpec((1, D), lambda b: (0, 0)),
            pl.BlockSpec((_DG, _W), lambda b: (0, 0)),
            pl.BlockSpec((1, D * N), lambda b: (0, 0)),
        ],
        out_specs=[
            pl.BlockSpec((_SB, K, N), lambda b: (b, 0, 0)),
            pl.BlockSpec((_SB, K, N), lambda b: (b, 0, 0)),
        ],
        out_shape=[
            jax.ShapeDtypeStruct((B, K, N), jnp.float32),
            jax.ShapeDtypeStruct((B, K, N), jnp.int32),
        ],
    )(xT, wlT, wrT, blT, brT, att_row, kron, catt)

    attention = vals.transpose(0, 2, 1).reshape(-1)
    index_j = idx.transpose(0, 2, 1).reshape(1, -1)
    index_i = (jnp.repeat(jnp.arange(N, dtype=jnp.int32), K)[None, :]
               + jnp.arange(B, dtype=jnp.int32)[:, None] * N).reshape(1, -1)
    new_edge_index = jnp.concatenate((index_i, index_j), axis=0)
    return new_edge_index, attention


# speedup vs baseline: 2.7886x; 1.1142x over previous
import jax
import jax.numpy as jnp
from jax import lax
from jax.experimental import pallas as pl
from jax.experimental.pallas import tpu as pltpu

B, N, C_IN, D, K = 64, 128, 64, 32, 20


_DG = 4
_NG = D // _DG
_W = _DG * N


_SB = 4


def _body(xT_ref, wlT_ref, wrT_ref, blT_ref, brT_ref, attv_ref, kron_ref,
          catt_ref, vals_ref, idx_ref):
    b = pl.program_id(0)
    for s in range(_SB):
        _sample(xT_ref[s], wlT_ref, wrT_ref, blT_ref, brT_ref, attv_ref,
                kron_ref, catt_ref, vals_ref, idx_ref, s, b * _SB + s)


def _sample(xTb, wlT_ref, wrT_ref, blT_ref, brT_ref, attv_ref, kron_ref,
            catt_ref, vals_ref, idx_ref, s, sample_idx):
    xlT = jnp.dot(wlT_ref[...], xTb, preferred_element_type=jnp.float32) + blT_ref[...]
    xrT = jnp.dot(wrT_ref[...], xTb, preferred_element_type=jnp.float32) + brT_ref[...]

    attv = attv_ref[...]
    ones_row = jnp.ones((1, N), dtype=jnp.float32)

    sr_row = 0.2 * jnp.dot(attv, xrT, preferred_element_type=jnp.float32)
    sl_row = 0.2 * jnp.dot(attv, xlT, preferred_element_type=jnp.float32)
    dn = (((0,), (0,)), ((), ()))
    acc0 = lax.dot_general(jnp.concatenate([sr_row, ones_row], axis=0),
                           jnp.concatenate([ones_row, sl_row], axis=0),
                           dn, preferred_element_type=jnp.float32)
    acc1 = jnp.zeros((N, N), dtype=jnp.float32)

    xlT_flat = xlT.reshape(1, D * N)
    kron = kron_ref[...]
    accs = [acc0, acc1]
    for g in range(_NG):
        p_g = jnp.concatenate([xrT[g * _DG:(g + 1) * _DG, :], ones_row], axis=0)
        q_g = jnp.concatenate([kron, xlT_flat[:, g * _W:(g + 1) * _W]], axis=0)
        s_g = lax.dot_general(p_g, q_g, dn, preferred_element_type=jnp.float32)
        t_g = jnp.maximum(s_g, 0.0) * catt_ref[:, g * _W:(g + 1) * _W]
        for t in range(_DG):
            accs[t % 2] = accs[t % 2] + t_g[:, t * N:(t + 1) * N]
    alpha = accs[0] + accs[1]

    alpha = jnp.where(jnp.isfinite(alpha), alpha, 0.0)

    _IMIN = jnp.int32(-2**31)
    _M31 = jnp.int32(0x7FFFFFFF)
    jiota = lax.broadcasted_iota(jnp.int32, (N, N), 0)
    u = lax.bitcast_convert_type(alpha, jnp.int32)
    m = u ^ ((u >> 31) & _M31)
    key = (m & jnp.int32(-128)) | (jnp.int32(127) - jiota)

    kmax_rows = []
    for _ in range(K):
        kmax = jnp.max(key, axis=0, keepdims=True)
        kmax_rows.append(kmax)
        key = jnp.where(key == kmax, _IMIN, key)

    kk = jnp.concatenate(kmax_rows, axis=0)
    jsel = jnp.int32(127) - (kk & jnp.int32(127))
    mt = kk & jnp.int32(-128)
    vals = lax.bitcast_convert_type(mt ^ ((mt >> 31) & _M31), jnp.float32)

    e = jnp.exp(vals - vals[0:1, :])
    vals_ref[s] = e / jnp.sum(e, axis=0, keepdims=True)
    idx_ref[s] = jsel + sample_idx * N


def kernel(x, edge_index, batch, W_l, b_l, W_r, b_r, att):
    del edge_index, batch
    xT = x.transpose(0, 2, 1)
    wlT = W_l.T
    wrT = W_r.T
    blT = b_l.reshape(D, 1)
    brT = b_r.reshape(D, 1)
    att_row = att.reshape(1, D)
    kron = jnp.kron(jnp.eye(_DG, dtype=jnp.float32),
                    jnp.ones((1, N), dtype=jnp.float32))
    catt = jnp.repeat(0.8 * att.reshape(-1), N)[None, :]

    vals, idx = pl.pallas_call(
        _body,
        grid=(B // _SB,),
        in_specs=[
            pl.BlockSpec((_SB, C_IN, N), lambda b: (b, 0, 0)),
            pl.BlockSpec((D, C_IN), lambda b: (0, 0)),
            pl.BlockSpec((D, C_IN), lambda b: (0, 0)),
            pl.BlockSpec((D, 1), lambda b: (0, 0)),
            pl.BlockSpec((D, 1), lambda b: (0, 0)),
            pl.BlockSpec((1, D), lambda b: (0, 0)),
            pl.BlockSpec((_DG, _W), lambda b: (0, 0)),
            pl.BlockSpec((1, D * N), lambda b: (0, 0)),
        ],
        out_specs=[
            pl.BlockSpec((_SB, K, N), lambda b: (b, 0, 0)),
            pl.BlockSpec((_SB, K, N), lambda b: (b, 0, 0)),
        ],
        out_shape=[
            jax.ShapeDtypeStruct((B, K, N), jnp.float32),
            jax.ShapeDtypeStruct((B, K, N), jnp.int32),
        ],
    )(xT, wlT, wrT, blT, brT, att_row, kron, catt)

    attention = vals.transpose(0, 2, 1).reshape(-1)
    index_j = idx.transpose(0, 2, 1).reshape(1, -1)
    index_i = (jnp.repeat(jnp.arange(N, dtype=jnp.int32), K)[None, :]
               + jnp.arange(B, dtype=jnp.int32)[:, None] * N).reshape(1, -1)
    new_edge_index = jnp.concatenate((index_i, index_j), axis=0)
    return new_edge_index, attention


# speedup vs baseline: 3.0369x; 1.0891x over previous
import jax
import jax.numpy as jnp
from jax import lax
from jax.experimental import pallas as pl
from jax.experimental.pallas import tpu as pltpu

B, N, C_IN, D, K = 64, 128, 64, 32, 20


_DG = 4
_NG = D // _DG
_W = _DG * N


_SB = 8


def _body(xT_ref, wlT_ref, wrT_ref, blT_ref, brT_ref, attv_ref, kron_ref,
          catt_ref, vals_ref, idx_ref):
    b = pl.program_id(0)
    for s in range(_SB):
        _sample(xT_ref[s], wlT_ref, wrT_ref, blT_ref, brT_ref, attv_ref,
                kron_ref, catt_ref, vals_ref, idx_ref, s, b * _SB + s)


def _sample(xTb, wlT_ref, wrT_ref, blT_ref, brT_ref, attv_ref, kron_ref,
            catt_ref, vals_ref, idx_ref, s, sample_idx):
    xlT = jnp.dot(wlT_ref[...], xTb, preferred_element_type=jnp.float32) + blT_ref[...]
    xrT = jnp.dot(wrT_ref[...], xTb, preferred_element_type=jnp.float32) + brT_ref[...]

    attv = attv_ref[...]
    ones_row = jnp.ones((1, N), dtype=jnp.float32)

    sr_row = 0.2 * jnp.dot(attv, xrT, preferred_element_type=jnp.float32)
    sl_row = 0.2 * jnp.dot(attv, xlT, preferred_element_type=jnp.float32)
    dn = (((0,), (0,)), ((), ()))
    acc0 = lax.dot_general(jnp.concatenate([sr_row, ones_row], axis=0),
                           jnp.concatenate([ones_row, sl_row], axis=0),
                           dn, preferred_element_type=jnp.float32)
    acc1 = jnp.zeros((N, N), dtype=jnp.float32)

    xlT_flat = xlT.reshape(1, D * N)
    kron = kron_ref[...]
    accs = [acc0, acc1]
    for g in range(_NG):
        p_g = jnp.concatenate([xrT[g * _DG:(g + 1) * _DG, :], ones_row], axis=0)
        q_g = jnp.concatenate([kron, xlT_flat[:, g * _W:(g + 1) * _W]], axis=0)
        s_g = lax.dot_general(p_g, q_g, dn, preferred_element_type=jnp.float32)
        t_g = jnp.maximum(s_g, 0.0) * catt_ref[:, g * _W:(g + 1) * _W]
        for t in range(_DG):
            accs[t % 2] = accs[t % 2] + t_g[:, t * N:(t + 1) * N]
    alpha = accs[0] + accs[1]

    alpha = jnp.where(jnp.isfinite(alpha), alpha, 0.0)

    _IMIN = jnp.int32(-2**31)
    _M31 = jnp.int32(0x7FFFFFFF)
    jiota = lax.broadcasted_iota(jnp.int32, (N, N), 0)
    u = lax.bitcast_convert_type(alpha, jnp.int32)
    m = u ^ ((u >> 31) & _M31)
    key = (m & jnp.int32(-128)) | (jnp.int32(127) - jiota)

    kmax_rows = []
    for _ in range(K):
        kmax = jnp.max(key, axis=0, keepdims=True)
        kmax_rows.append(kmax)
        key = jnp.where(key == kmax, _IMIN, key)

    kk = jnp.concatenate(kmax_rows, axis=0)
    jsel = jnp.int32(127) - (kk & jnp.int32(127))
    mt = kk & jnp.int32(-128)
    vals = lax.bitcast_convert_type(mt ^ ((mt >> 31) & _M31), jnp.float32)

    e = jnp.exp(vals - vals[0:1, :])
    vals_ref[s] = e / jnp.sum(e, axis=0, keepdims=True)
    idx_ref[s] = jsel + sample_idx * N


def kernel(x, edge_index, batch, W_l, b_l, W_r, b_r, att):
    del edge_index, batch
    xT = x.transpose(0, 2, 1)
    wlT = W_l.T
    wrT = W_r.T
    blT = b_l.reshape(D, 1)
    brT = b_r.reshape(D, 1)
    att_row = att.reshape(1, D)
    kron = jnp.kron(jnp.eye(_DG, dtype=jnp.float32),
                    jnp.ones((1, N), dtype=jnp.float32))
    catt = jnp.repeat(0.8 * att.reshape(-1), N)[None, :]

    vals, idx = pl.pallas_call(
        _body,
        grid=(B // _SB,),
        in_specs=[
            pl.BlockSpec((_SB, C_IN, N), lambda b: (b, 0, 0)),
            pl.BlockSpec((D, C_IN), lambda b: (0, 0)),
            pl.BlockSpec((D, C_IN), lambda b: (0, 0)),
            pl.BlockSpec((D, 1), lambda b: (0, 0)),
            pl.BlockSpec((D, 1), lambda b: (0, 0)),
            pl.BlockSpec((1, D), lambda b: (0, 0)),
            pl.BlockSpec((_DG, _W), lambda b: (0, 0)),
            pl.BlockSpec((1, D * N), lambda b: (0, 0)),
        ],
        out_specs=[
            pl.BlockSpec((_SB, K, N), lambda b: (b, 0, 0)),
            pl.BlockSpec((_SB, K, N), lambda b: (b, 0, 0)),
        ],
        out_shape=[
            jax.ShapeDtypeStruct((B, K, N), jnp.float32),
            jax.ShapeDtypeStruct((B, K, N), jnp.int32),
        ],
    )(xT, wlT, wrT, blT, brT, att_row, kron, catt)

    attention = vals.transpose(0, 2, 1).reshape(-1)
    index_j = idx.transpose(0, 2, 1).reshape(1, -1)
    index_i = (jnp.repeat(jnp.arange(N, dtype=jnp.int32), K)[None, :]
               + jnp.arange(B, dtype=jnp.int32)[:, None] * N).reshape(1, -1)
    new_edge_index = jnp.concatenate((index_i, index_j), axis=0)
    return new_edge_index, attention


# speedup vs baseline: 3.1798x; 1.0471x over previous
import jax
import jax.numpy as jnp
from jax import lax
from jax.experimental import pallas as pl
from jax.experimental.pallas import tpu as pltpu

B, N, C_IN, D, K = 64, 128, 64, 32, 20


_DG = 4
_NG = D // _DG
_W = _DG * N


_SB = 16


def _body(xT_ref, wlT_ref, wrT_ref, blT_ref, brT_ref, attv_ref, kron_ref,
          catt_ref, vals_ref, idx_ref):
    b = pl.program_id(0)
    for s in range(_SB):
        _sample(xT_ref[s], wlT_ref, wrT_ref, blT_ref, brT_ref, attv_ref,
                kron_ref, catt_ref, vals_ref, idx_ref, s, b * _SB + s)


def _sample(xTb, wlT_ref, wrT_ref, blT_ref, brT_ref, attv_ref, kron_ref,
            catt_ref, vals_ref, idx_ref, s, sample_idx):
    xlT = jnp.dot(wlT_ref[...], xTb, preferred_element_type=jnp.float32) + blT_ref[...]
    xrT = jnp.dot(wrT_ref[...], xTb, preferred_element_type=jnp.float32) + brT_ref[...]

    attv = attv_ref[...]
    ones_row = jnp.ones((1, N), dtype=jnp.float32)

    sr_row = 0.2 * jnp.dot(attv, xrT, preferred_element_type=jnp.float32)
    sl_row = 0.2 * jnp.dot(attv, xlT, preferred_element_type=jnp.float32)
    dn = (((0,), (0,)), ((), ()))
    acc0 = lax.dot_general(jnp.concatenate([sr_row, ones_row], axis=0),
                           jnp.concatenate([ones_row, sl_row], axis=0),
                           dn, preferred_element_type=jnp.float32)
    acc1 = jnp.zeros((N, N), dtype=jnp.float32)

    xlT_flat = xlT.reshape(1, D * N)
    kron = kron_ref[...]
    accs = [acc0, acc1]
    for g in range(_NG):
        p_g = jnp.concatenate([xrT[g * _DG:(g + 1) * _DG, :], ones_row], axis=0)
        q_g = jnp.concatenate([kron, xlT_flat[:, g * _W:(g + 1) * _W]], axis=0)
        s_g = lax.dot_general(p_g, q_g, dn, preferred_element_type=jnp.float32)
        t_g = jnp.maximum(s_g, 0.0) * catt_ref[:, g * _W:(g + 1) * _W]
        for t in range(_DG):
            accs[t % 2] = accs[t % 2] + t_g[:, t * N:(t + 1) * N]
    alpha = accs[0] + accs[1]

    alpha = jnp.where(jnp.isfinite(alpha), alpha, 0.0)

    _IMIN = jnp.int32(-2**31)
    _M31 = jnp.int32(0x7FFFFFFF)
    jiota = lax.broadcasted_iota(jnp.int32, (N, N), 0)
    u = lax.bitcast_convert_type(alpha, jnp.int32)
    m = u ^ ((u >> 31) & _M31)
    key = (m & jnp.int32(-128)) | (jnp.int32(127) - jiota)

    kmax_rows = []
    for _ in range(K):
        kmax = jnp.max(key, axis=0, keepdims=True)
        kmax_rows.append(kmax)
        key = jnp.where(key == kmax, _IMIN, key)

    kk = jnp.concatenate(kmax_rows, axis=0)
    jsel = jnp.int32(127) - (kk & jnp.int32(127))
    mt = kk & jnp.int32(-128)
    vals = lax.bitcast_convert_type(mt ^ ((mt >> 31) & _M31), jnp.float32)

    e = jnp.exp(vals - vals[0:1, :])
    vals_ref[s] = e / jnp.sum(e, axis=0, keepdims=True)
    idx_ref[s] = jsel + sample_idx * N


def kernel(x, edge_index, batch, W_l, b_l, W_r, b_r, att):
    del edge_index, batch
    xT = x.transpose(0, 2, 1)
    wlT = W_l.T
    wrT = W_r.T
    blT = b_l.reshape(D, 1)
    brT = b_r.reshape(D, 1)
    att_row = att.reshape(1, D)
    kron = jnp.kron(jnp.eye(_DG, dtype=jnp.float32),
                    jnp.ones((1, N), dtype=jnp.float32))
    catt = jnp.repeat(0.8 * att.reshape(-1), N)[None, :]

    vals, idx = pl.pallas_call(
        _body,
        grid=(B // _SB,),
        in_specs=[
            pl.BlockSpec((_SB, C_IN, N), lambda b: (b, 0, 0)),
            pl.BlockSpec((D, C_IN), lambda b: (0, 0)),
            pl.BlockSpec((D, C_IN), lambda b: (0, 0)),
            pl.BlockSpec((D, 1), lambda b: (0, 0)),
            pl.BlockSpec((D, 1), lambda b: (0, 0)),
            pl.BlockSpec((1, D), lambda b: (0, 0)),
            pl.BlockSpec((_DG, _W), lambda b: (0, 0)),
            pl.BlockSpec((1, D * N), lambda b: (0, 0)),
        ],
        out_specs=[
            pl.BlockSpec((_SB, K, N), lambda b: (b, 0, 0)),
            pl.BlockSpec((_SB, K, N), lambda b: (b, 0, 0)),
        ],
        out_shape=[
            jax.ShapeDtypeStruct((B, K, N), jnp.float32),
            jax.ShapeDtypeStruct((B, K, N), jnp.int32),
        ],
    )(xT, wlT, wrT, blT, brT, att_row, kron, catt)

    attention = vals.transpose(0, 2, 1).reshape(-1)
    index_j = idx.transpose(0, 2, 1).reshape(1, -1)
    index_i = (jnp.repeat(jnp.arange(N, dtype=jnp.int32), K)[None, :]
               + jnp.arange(B, dtype=jnp.int32)[:, None] * N).reshape(1, -1)
    new_edge_index = jnp.concatenate((index_i, index_j), axis=0)
    return new_edge_index, attention
